# baseline (speedup 1.0000x reference)
import functools

import jax
import jax.numpy as jnp
from jax import lax
from jax.experimental import pallas as pl
from jax.experimental.pallas import tpu as pltpu
from jax.experimental.pallas import tpu_sc as plsc

HIDDEN = 768
INTER = 768
NUM_EXPERTS = 16
TOP_K = 2
TOKENS = 32
SWIGLU_LIMIT = 7.0
ALPHA = 1.702
EPS = 1e-5


def _tc_head_kernel(x_ref, scale_ref, gw_ref, gb_ref, t_ref, g_ref):
    x = x_ref[...]
    ms = jnp.mean(x * x, axis=1, keepdims=True)
    t = x * lax.rsqrt(ms + EPS) * scale_ref[...]
    g = lax.dot_general(t, gw_ref[...], (((1,), (1,)), ((), ())),
                        preferred_element_type=jnp.float32) + gb_ref[...]
    t_ref[...] = t
    g_ref[...] = g


def _sc_router_body(g_hbm, c_hbm, gv, cv):
    wid = lax.axis_index("s") * 2 + lax.axis_index("c")
    pltpu.sync_copy(g_hbm.at[wid], gv)
    v = gv[...]
    ii = lax.iota(jnp.int32, NUM_EXPERTS)
    m1 = jnp.max(v)
    i1 = jnp.min(jnp.where(v == m1, ii, NUM_EXPERTS))
    v2 = jnp.where(ii == i1, -jnp.inf, v)
    m2 = jnp.max(v2)
    i2 = jnp.min(jnp.where(v2 == m2, ii, NUM_EXPERTS))
    b = jnp.exp(jnp.broadcast_to(m2 - m1, (NUM_EXPERTS,)))
    w1 = 1.0 / (1.0 + b)
    w2 = b / (1.0 + b)
    cv[...] = jnp.where(ii == i1, w1, 0.0) + jnp.where(ii == i2, w2, 0.0)
    pltpu.sync_copy(cv, c_hbm.at[wid])


def _tc_sweep_kernel(t_ref, w1_ref, b1_ref, w2_ref, b2_ref, y_ref, s_s):
    e = pl.program_id(0)

    @pl.when(e == 0)
    def _():
        rows = lax.broadcasted_iota(jnp.int32, (256, 128), 0)
        cols = lax.broadcasted_iota(jnp.int32, (256, 128), 1)
        s_s[...] = jnp.where(rows == 2 * cols, 1.0, 0.0)

    t = t_ref[...]
    h = lax.dot_general(t, w1_ref[0], (((1,), (1,)), ((), ())),
                        preferred_element_type=jnp.float32)
    h = h + b1_ref[pl.ds(e, 1), :]
    hs = pltpu.roll(h, 2 * INTER - 1, 1)
    hg = jnp.minimum(h, SWIGLU_LIMIT)
    hl = jnp.clip(hs, -SWIGLU_LIMIT, SWIGLU_LIMIT)
    v = hg * jax.nn.sigmoid(ALPHA * hg) * (hl + 1.0)
    sb = s_s[...]
    act = jnp.concatenate(
        [lax.dot_general(v[:, 256 * j:256 * (j + 1)], sb,
                         (((1,), (0,)), ((), ())),
                         preferred_element_type=jnp.float32)
         for j in range(2 * INTER // 256)], axis=1)
    y = lax.dot_general(act, w2_ref[0], (((1,), (1,)), ((), ())),
                        preferred_element_type=jnp.float32)
    y_ref[0] = y + b2_ref[pl.ds(e, 1), :]


def _tc_combine_kernel(x_ref, c_ref, y_ref, o_ref):
    acc = x_ref[...]
    c = c_ref[...]
    ii = lax.broadcasted_iota(jnp.int32, (TOKENS, NUM_EXPERTS), 1)
    for e in range(NUM_EXPERTS):
        ce = jnp.sum(c * jnp.where(ii == e, 1.0, 0.0), axis=1, keepdims=True)
        acc = acc + ce * y_ref[e]
    o_ref[...] = acc


@jax.jit
def kernel(x, norm_scale, gate_w, gate_b, mlp1_w, mlp1_b, mlp2_w, mlp2_b):
    t, g = pl.pallas_call(
        _tc_head_kernel,
        out_shape=(
            jax.ShapeDtypeStruct((TOKENS, HIDDEN), jnp.float32),
            jax.ShapeDtypeStruct((TOKENS, NUM_EXPERTS), jnp.float32),
        ),
    )(x, norm_scale.reshape(1, HIDDEN), gate_w, gate_b.reshape(1, NUM_EXPERTS))

    sc_router = functools.partial(
        pl.kernel,
        out_type=jax.ShapeDtypeStruct((TOKENS, NUM_EXPERTS), jnp.float32),
        mesh=plsc.VectorSubcoreMesh(core_axis_name="c", subcore_axis_name="s"),
        scratch_types=[
            pltpu.VMEM((NUM_EXPERTS,), jnp.float32),
            pltpu.VMEM((NUM_EXPERTS,), jnp.float32),
        ],
        compiler_params=pltpu.CompilerParams(needs_layout_passes=False),
    )(_sc_router_body)
    c = sc_router(g)

    y_all = pl.pallas_call(
        _tc_sweep_kernel,
        grid=(NUM_EXPERTS,),
        in_specs=[
            pl.BlockSpec((TOKENS, HIDDEN), lambda e: (0, 0)),
            pl.BlockSpec((1, 2 * INTER, HIDDEN), lambda e: (e, 0, 0)),
            pl.BlockSpec((NUM_EXPERTS, 2 * INTER), lambda e: (0, 0)),
            pl.BlockSpec((1, HIDDEN, INTER), lambda e: (e, 0, 0)),
            pl.BlockSpec((NUM_EXPERTS, HIDDEN), lambda e: (0, 0)),
        ],
        out_specs=pl.BlockSpec((1, TOKENS, HIDDEN), lambda e: (e, 0, 0)),
        out_shape=jax.ShapeDtypeStruct((NUM_EXPERTS, TOKENS, HIDDEN),
                                       jnp.float32),
        scratch_shapes=[
            pltpu.VMEM((256, 128), jnp.float32),
        ],
        compiler_params=pltpu.CompilerParams(
            dimension_semantics=("arbitrary",),
        ),
    )(t, mlp1_w, mlp1_b, mlp2_w, mlp2_b)

    out = pl.pallas_call(
        _tc_combine_kernel,
        out_shape=jax.ShapeDtypeStruct((TOKENS, HIDDEN), jnp.float32),
    )(x, c, y_all)
    return out


# speedup vs baseline: 1.4114x; 1.4114x over previous
import jax
import jax.numpy as jnp
from jax import lax
from jax.experimental import pallas as pl
from jax.experimental.pallas import tpu as pltpu

HIDDEN = 768
INTER = 768
NUM_EXPERTS = 16
TOP_K = 2
TOKENS = 32
SWIGLU_LIMIT = 7.0
ALPHA = 1.702
EPS = 1e-5


def _moe_kernel(x_ref, scale_ref, gw_ref, gb_ref, w1_ref, b1_ref, w2_ref,
                b2_ref, o_ref, t_s, c_s, s_s):
    e = pl.program_id(0)

    @pl.when(e == 0)
    def _():
        x = x_ref[...]
        ms = jnp.mean(x * x, axis=1, keepdims=True)
        t = x * lax.rsqrt(ms + EPS) * scale_ref[...]
        g = lax.dot_general(t, gw_ref[...], (((1,), (1,)), ((), ())),
                            preferred_element_type=jnp.float32) + gb_ref[...]
        ii = lax.broadcasted_iota(jnp.int32, (TOKENS, NUM_EXPERTS), 1)
        m1 = jnp.max(g, axis=1, keepdims=True)
        i1 = jnp.min(jnp.where(g == m1, ii, NUM_EXPERTS), axis=1,
                     keepdims=True)
        g2 = jnp.where(ii == i1, -jnp.inf, g)
        m2 = jnp.max(g2, axis=1, keepdims=True)
        i2 = jnp.min(jnp.where(g2 == m2, ii, NUM_EXPERTS), axis=1,
                     keepdims=True)
        b = jnp.exp(m2 - m1)
        w1 = 1.0 / (1.0 + b)
        w2 = b / (1.0 + b)
        t_s[...] = t
        c_s[...] = jnp.where(ii == i1, w1, 0.0) + jnp.where(ii == i2, w2, 0.0)
        rows = lax.broadcasted_iota(jnp.int32, (256, 256), 0)
        cols = lax.broadcasted_iota(jnp.int32, (256, 256), 1)
        s_s[...] = jnp.where((rows == 2 * cols) |
                             (rows == 2 * (cols - 128) + 1), 1.0, 0.0)

    t = t_s[...]
    h = lax.dot_general(t, w1_ref[0], (((1,), (1,)), ((), ())),
                        preferred_element_type=jnp.float32)
    h = h + b1_ref[pl.ds(e, 1), :]
    sb = s_s[...]
    acts = []
    for j in range(2 * INTER // 256):
        hc = lax.dot_general(h[:, 256 * j:256 * (j + 1)], sb,
                             (((1,), (0,)), ((), ())),
                             preferred_element_type=jnp.float32)
        hg = jnp.minimum(hc[:, :128], SWIGLU_LIMIT)
        hl = jnp.clip(hc[:, 128:], -SWIGLU_LIMIT, SWIGLU_LIMIT)
        acts.append(hg * jax.nn.sigmoid(ALPHA * hg) * (hl + 1.0))
    act = jnp.concatenate(acts, axis=1)
    y = lax.dot_general(act, w2_ref[0], (((1,), (1,)), ((), ())),
                        preferred_element_type=jnp.float32)
    y = y + b2_ref[pl.ds(e, 1), :]
    ii = lax.broadcasted_iota(jnp.int32, (TOKENS, NUM_EXPERTS), 1)
    ce = jnp.sum(c_s[...] * jnp.where(ii == e, 1.0, 0.0), axis=1,
                 keepdims=True)
    contrib = ce * y

    @pl.when(e == 0)
    def _():
        o_ref[...] = x_ref[...] + contrib

    @pl.when(e != 0)
    def _():
        o_ref[...] += contrib


@jax.jit
def kernel(x, norm_scale, gate_w, gate_b, mlp1_w, mlp1_b, mlp2_w, mlp2_b):
    return pl.pallas_call(
        _moe_kernel,
        grid=(NUM_EXPERTS,),
        in_specs=[
            pl.BlockSpec((TOKENS, HIDDEN), lambda e: (0, 0)),
            pl.BlockSpec((1, HIDDEN), lambda e: (0, 0)),
            pl.BlockSpec((NUM_EXPERTS, HIDDEN), lambda e: (0, 0)),
            pl.BlockSpec((1, NUM_EXPERTS), lambda e: (0, 0)),
            pl.BlockSpec((1, 2 * INTER, HIDDEN), lambda e: (e, 0, 0)),
            pl.BlockSpec((NUM_EXPERTS, 2 * INTER), lambda e: (0, 0)),
            pl.BlockSpec((1, HIDDEN, INTER), lambda e: (e, 0, 0)),
            pl.BlockSpec((NUM_EXPERTS, HIDDEN), lambda e: (0, 0)),
        ],
        out_specs=pl.BlockSpec((TOKENS, HIDDEN), lambda e: (0, 0)),
        out_shape=jax.ShapeDtypeStruct((TOKENS, HIDDEN), jnp.float32),
        scratch_shapes=[
            pltpu.VMEM((TOKENS, HIDDEN), jnp.float32),
            pltpu.VMEM((TOKENS, NUM_EXPERTS), jnp.float32),
            pltpu.VMEM((256, 256), jnp.float32),
        ],
        compiler_params=pltpu.CompilerParams(
            dimension_semantics=("arbitrary",),
        ),
    )(x, norm_scale.reshape(1, HIDDEN), gate_w, gate_b.reshape(1, NUM_EXPERTS),
      mlp1_w, mlp1_b, mlp2_w, mlp2_b)
